# baseline (speedup 1.0000x reference)
import functools

import jax
import jax.numpy as jnp
from jax import lax
from jax.experimental import pallas as pl
from jax.experimental.pallas import tpu as pltpu
from jax.experimental.pallas import tpu_sc as plsc

VOCAB = 128100
EMB = 512
HID = 1024
B = 4
S = 2048
EPS = 1e-07

N_TOK = B * S
EMBW = EMB // 2

_CHUNK = 64
_GRP = EMB // 32


def _pack_chunk(rows_ref, out_ref):
    half = jnp.full((16,), 0x8000, dtype=jnp.int32)
    himask = jnp.full((16,), -65536, dtype=jnp.int32)
    sh16 = jnp.full((16,), 16, dtype=jnp.int32)

    def grp_body(g, carry):
        colb = pl.multiple_of(g * 32, 32)
        colw = pl.multiple_of(g * 16, 16)
        for r in range(_CHUNK):
            a = rows_ref[r, pl.ds(colb, 16)]
            b = rows_ref[r, pl.ds(colb + 16, 16)]
            ra = lax.bitcast_convert_type(a, jnp.int32) + half
            rb = lax.bitcast_convert_type(b, jnp.int32) + half
            out_ref[r, pl.ds(colw, 16)] = (
                (rb & himask) | lax.shift_right_logical(ra, sh16))
        return carry

    lax.fori_loop(0, _GRP, grp_body, 0)


def _make_sc_gather():
    info = plsc.get_sparse_core_info()
    nc, ns = info.num_cores, info.num_subcores
    nw = nc * ns
    per_w = N_TOK // nw
    n_chunks = per_w // _CHUNK
    w_per_row = S // per_w
    mesh = plsc.VectorSubcoreMesh(core_axis_name="c", subcore_axis_name="s")

    @functools.partial(
        pl.kernel,
        mesh=mesh,
        out_type=jax.ShapeDtypeStruct((N_TOK, EMBW), jnp.int32),
        scratch_types=[
            pltpu.VMEM((2, _CHUNK), jnp.int32),
            pltpu.VMEM((_CHUNK, EMB), jnp.float32),
            pltpu.VMEM((_CHUNK, EMB), jnp.float32),
            pltpu.VMEM((_CHUNK, EMBW), jnp.int32),
            pltpu.VMEM((_CHUNK, EMBW), jnp.int32),
            pltpu.SemaphoreType.DMA,
            pltpu.SemaphoreType.DMA,
            pltpu.SemaphoreType.DMA,
            pltpu.SemaphoreType.DMA,
        ],
    )
    def gather_k(idx_hbm, table_hbm, out_hbm, idx_v, rows0, rows1,
                 opk0, opk1, g0, g1, o0, o1):
        rows = (rows0, rows1)
        opk = (opk0, opk1)
        gsem = (g0, g1)
        osem = (o0, o1)
        wid = lax.axis_index("s") * nc + lax.axis_index("c")
        row = wid // w_per_row
        col0 = (wid % w_per_row) * per_w
        base0 = wid * per_w

        def idx_load(c, b):
            pltpu.sync_copy(idx_hbm.at[row, pl.ds(col0 + c * _CHUNK, _CHUNK)],
                            idx_v.at[b])

        def gather_start(c, b):
            return pltpu.async_copy(table_hbm.at[idx_v.at[b]], rows[b],
                                    gsem[b])

        def out_start(c, b):
            return pltpu.async_copy(
                opk[b], out_hbm.at[pl.ds(base0 + c * _CHUNK, _CHUNK)],
                osem[b])

        gh = [None] * n_chunks
        oh = [None] * n_chunks
        for b in range(2):
            idx_load(b, b)
            gh[b] = gather_start(b, b)
        for c in range(n_chunks):
            b = c % 2
            gh[c].wait()
            if c >= 2:
                oh[c - 2].wait()
            _pack_chunk(rows[b], opk[b])
            oh[c] = out_start(c, b)
            nxt = c + 2
            if nxt < n_chunks:
                idx_load(nxt, b)
                gh[nxt] = gather_start(nxt, b)
        oh[n_chunks - 2].wait()
        oh[n_chunks - 1].wait()

    return gather_k


def _tc_body(g_ref, plo_ref, phi_ref, wlo_ref, whi_ref,
             gamma_ref, beta_ref, o_ref):
    word = g_ref[...]
    lo = lax.bitcast_convert_type(jnp.left_shift(word, 16), jnp.float32)
    hi = lax.bitcast_convert_type(word & jnp.int32(-65536), jnp.float32)
    xlo = (lo + plo_ref[...]).astype(jnp.bfloat16)
    xhi = (hi + phi_ref[...]).astype(jnp.bfloat16)
    dn = (((1,), (1,)), ((), ()))
    y = lax.dot_general(xlo, wlo_ref[...].astype(jnp.bfloat16), dn,
                        preferred_element_type=jnp.float32)
    y = y + lax.dot_general(xhi, whi_ref[...].astype(jnp.bfloat16), dn,
                            preferred_element_type=jnp.float32)
    mean = jnp.mean(y, axis=-1, keepdims=True)
    yc = y - mean
    var = jnp.mean(yc * yc, axis=-1, keepdims=True)
    o_ref[0] = yc * lax.rsqrt(var + EPS) * gamma_ref[...] + beta_ref[...]


def _tc_call(packed, plo, phi, wlo, whi, gamma, beta):
    return pl.pallas_call(
        _tc_body,
        grid=(B,),
        in_specs=[
            pl.BlockSpec((S, EMBW), lambda j: (j, 0)),
            pl.BlockSpec((S, EMBW), lambda j: (0, 0)),
            pl.BlockSpec((S, EMBW), lambda j: (0, 0)),
            pl.BlockSpec((HID, EMBW), lambda j: (0, 0)),
            pl.BlockSpec((HID, EMBW), lambda j: (0, 0)),
            pl.BlockSpec((1, HID), lambda j: (0, 0)),
            pl.BlockSpec((1, HID), lambda j: (0, 0)),
        ],
        out_specs=pl.BlockSpec((1, S, HID), lambda j: (j, 0, 0)),
        out_shape=jax.ShapeDtypeStruct((B, S, HID), jnp.float32),
    )(packed, plo, phi, wlo, whi, gamma, beta)


def kernel(input_ids, word_embeddings, position_embeddings, proj_weight, ln_gamma, ln_beta):
    packed = _make_sc_gather()(input_ids, word_embeddings)
    pr = position_embeddings.reshape(S, _GRP, 2, 16)
    plo = pr[:, :, 0, :].reshape(S, EMBW)
    phi = pr[:, :, 1, :].reshape(S, EMBW)
    wr = proj_weight.reshape(HID, _GRP, 2, 16)
    wlo = wr[:, :, 0, :].reshape(HID, EMBW)
    whi = wr[:, :, 1, :].reshape(HID, EMBW)
    return _tc_call(packed, plo, phi, wlo, whi,
                    ln_gamma.reshape(1, HID), ln_beta.reshape(1, HID))


# speedup vs baseline: 1.3548x; 1.3548x over previous
import functools

import jax
import jax.numpy as jnp
from jax import lax
from jax.experimental import pallas as pl
from jax.experimental.pallas import tpu as pltpu
from jax.experimental.pallas import tpu_sc as plsc

VOCAB = 128100
EMB = 512
HID = 1024
B = 4
S = 2048
EPS = 1e-07

N_TOK = B * S

_CHUNK = 128


def _make_sc_gather():
    info = plsc.get_sparse_core_info()
    nc, ns = info.num_cores, info.num_subcores
    nw = nc * ns
    per_w = N_TOK // nw
    n_chunks = per_w // _CHUNK
    mesh = plsc.VectorSubcoreMesh(core_axis_name="c", subcore_axis_name="s")

    @functools.partial(
        pl.kernel,
        mesh=mesh,
        out_type=jax.ShapeDtypeStruct((N_TOK, EMB), jnp.float32),
        scratch_types=[
            pltpu.VMEM((_CHUNK,), jnp.int32),
            pltpu.VMEM((_CHUNK, EMB), jnp.float32),
            pltpu.SemaphoreType.DMA,
        ],
    )
    def gather_k(idx_hbm, table_hbm, out_hbm, idx_v, rows_v, sem):
        wid = lax.axis_index("s") * nc + lax.axis_index("c")
        base0 = wid * per_w
        for c in range(n_chunks):
            base = base0 + c * _CHUNK
            pltpu.sync_copy(idx_hbm.at[pl.ds(base, _CHUNK)], idx_v)
            pltpu.async_copy(table_hbm.at[idx_v], rows_v, sem).wait()
            pltpu.sync_copy(rows_v, out_hbm.at[pl.ds(base, _CHUNK)])

    return gather_k


_BLK = 2048


def _tc_body(g_ref, p_ref, w_ref, gamma_ref, beta_ref, o_ref):
    x = (g_ref[...] + p_ref[...]).astype(jnp.bfloat16)
    y = lax.dot_general(x, w_ref[...].astype(jnp.bfloat16),
                        (((1,), (1,)), ((), ())),
                        preferred_element_type=jnp.float32)
    mean = jnp.mean(y, axis=-1, keepdims=True)
    yc = y - mean
    var = jnp.mean(yc * yc, axis=-1, keepdims=True)
    o_ref[...] = yc * lax.rsqrt(var + EPS) * gamma_ref[...] + beta_ref[...]


def _tc_call(gathered, pos, w, gamma, beta):
    s_blocks = S // _BLK
    return pl.pallas_call(
        _tc_body,
        grid=(s_blocks, B),
        in_specs=[
            pl.BlockSpec((_BLK, EMB), lambda i, j: (j * s_blocks + i, 0)),
            pl.BlockSpec((_BLK, EMB), lambda i, j: (i, 0)),
            pl.BlockSpec((HID, EMB), lambda i, j: (0, 0)),
            pl.BlockSpec((1, HID), lambda i, j: (0, 0)),
            pl.BlockSpec((1, HID), lambda i, j: (0, 0)),
        ],
        out_specs=pl.BlockSpec((_BLK, HID), lambda i, j: (j * s_blocks + i, 0)),
        out_shape=jax.ShapeDtypeStruct((N_TOK, HID), jnp.float32),
    )(gathered, pos, w, gamma, beta)


def kernel(input_ids, word_embeddings, position_embeddings, proj_weight, ln_gamma, ln_beta):
    ids_flat = input_ids.reshape(N_TOK)
    gathered = _make_sc_gather()(ids_flat, word_embeddings)
    out = _tc_call(
        gathered,
        position_embeddings,
        proj_weight,
        ln_gamma.reshape(1, HID),
        ln_beta.reshape(1, HID),
    )
    return out.reshape(B, S, HID)
